# Initial kernel scaffold; baseline (speedup 1.0000x reference)
import jax
import jax.numpy as jnp
from jax.experimental import pallas as pl


def kernel(inputs, table, ln_weight):
    raise NotImplementedError("write your pallas kernel here")


# speedup vs baseline: 2.9963x; 2.9963x over previous
import jax
import jax.numpy as jnp
from jax.experimental import pallas as pl
from jax.experimental.pallas import tpu as pltpu

HIDDEN = 1024
EPS = 1e-6
BLOCK_ROWS = 512


def _rmsnorm_body(x_ref, w_ref, o_ref):
    x = x_ref[...]
    var = jnp.mean(x * x, axis=-1, keepdims=True)
    o_ref[...] = x * jax.lax.rsqrt(var + EPS) * w_ref[...]


def kernel(inputs, table, ln_weight):
    seq_len = inputs.shape[1]
    rows = table[:seq_len]
    n_blocks = seq_len // BLOCK_ROWS
    w2d = ln_weight.reshape(1, HIDDEN)
    out = pl.pallas_call(
        _rmsnorm_body,
        grid=(n_blocks,),
        in_specs=[
            pl.BlockSpec((BLOCK_ROWS, HIDDEN), lambda i: (i, 0)),
            pl.BlockSpec((1, HIDDEN), lambda i: (0, 0)),
        ],
        out_specs=pl.BlockSpec((BLOCK_ROWS, HIDDEN), lambda i: (i, 0)),
        out_shape=jax.ShapeDtypeStruct((seq_len, HIDDEN), jnp.float32),
        compiler_params=pltpu.CompilerParams(
            dimension_semantics=("parallel",),
        ),
    )(rows, w2d)
    return out[jnp.newaxis]


# speedup vs baseline: 3.4857x; 1.1633x over previous
import jax
import jax.numpy as jnp
from jax.experimental import pallas as pl
from jax.experimental.pallas import tpu as pltpu

HIDDEN = 1024
EPS = 1e-6
BLOCK_ROWS = 2048


def _rmsnorm_body(x_ref, w_ref, o_ref):
    x = x_ref[...]
    var = jnp.mean(x * x, axis=-1, keepdims=True)
    o_ref[...] = x * jax.lax.rsqrt(var + EPS) * w_ref[...]


def kernel(inputs, table, ln_weight):
    seq_len = inputs.shape[1]
    rows = table[:seq_len]
    n_blocks = seq_len // BLOCK_ROWS
    w2d = ln_weight.reshape(1, HIDDEN)
    out = pl.pallas_call(
        _rmsnorm_body,
        grid=(n_blocks,),
        in_specs=[
            pl.BlockSpec((BLOCK_ROWS, HIDDEN), lambda i: (i, 0)),
            pl.BlockSpec((1, HIDDEN), lambda i: (0, 0)),
        ],
        out_specs=pl.BlockSpec((BLOCK_ROWS, HIDDEN), lambda i: (i, 0)),
        out_shape=jax.ShapeDtypeStruct((seq_len, HIDDEN), jnp.float32),
        compiler_params=pltpu.CompilerParams(
            dimension_semantics=("parallel",),
        ),
    )(rows, w2d)
    return out[jnp.newaxis]
